# baseline (speedup 1.0000x reference)
import jax
import jax.numpy as jnp
from jax.experimental import pallas as pl
from jax.experimental.pallas import tpu as pltpu


def _expert_ffn_kernel(x_ref, w1_ref, b1_ref, w2_ref, b2_ref, o_ref):
    ep, cap, d = x_ref.shape
    x = x_ref[...].reshape(ep * cap, d).astype(jnp.bfloat16)
    w1 = w1_ref[0].astype(jnp.bfloat16)
    h = jnp.dot(x, w1, preferred_element_type=jnp.float32)
    h = jax.nn.gelu(h + b1_ref[0])
    w2 = w2_ref[0].astype(jnp.bfloat16)
    o = jnp.dot(h.astype(jnp.bfloat16), w2, preferred_element_type=jnp.float32)
    o = o + b2_ref[0]
    o_ref[...] = o.reshape(ep, cap, d)


def kernel(inputs, W1, b1, W2, b2):
    ep, n, d = inputs.shape
    e, _, d_ff = W1.shape
    cap = n // e
    b1 = b1.reshape(e, 1, d_ff)
    b2 = b2.reshape(e, 1, d)

    grid = (e,)
    return pl.pallas_call(
        _expert_ffn_kernel,
        grid=grid,
        in_specs=[
            pl.BlockSpec((ep, cap, d), lambda i: (0, i, 0)),
            pl.BlockSpec((1, d, d_ff), lambda i: (i, 0, 0)),
            pl.BlockSpec((1, 1, d_ff), lambda i: (i, 0, 0)),
            pl.BlockSpec((1, d_ff, d), lambda i: (i, 0, 0)),
            pl.BlockSpec((1, 1, d), lambda i: (i, 0, 0)),
        ],
        out_specs=pl.BlockSpec((ep, cap, d), lambda i: (0, i, 0)),
        out_shape=jax.ShapeDtypeStruct((ep, n, d), jnp.float32),
    )(inputs, W1, b1, W2, b2)


# speedup vs baseline: 1.0230x; 1.0230x over previous
import jax
import jax.numpy as jnp
from jax.experimental import pallas as pl
from jax.experimental.pallas import tpu as pltpu


def _expert_ffn_kernel(x_ref, w1_ref, b1_ref, w2_ref, b2_ref, o_ref):
    ep, cap, d = x_ref.shape
    x = x_ref[...].reshape(ep * cap, d)
    h = jnp.dot(x, w1_ref[0], preferred_element_type=jnp.float32)
    h = jax.nn.gelu(h + b1_ref[0])
    o = jnp.dot(h, w2_ref[0], preferred_element_type=jnp.float32)
    o = o + b2_ref[0]
    o_ref[...] = o.reshape(ep, cap, d)


def kernel(inputs, W1, b1, W2, b2):
    ep, n, d = inputs.shape
    e, _, d_ff = W1.shape
    cap = n // e
    b1 = b1.reshape(e, 1, d_ff)
    b2 = b2.reshape(e, 1, d)

    grid = (e,)
    return pl.pallas_call(
        _expert_ffn_kernel,
        grid=grid,
        in_specs=[
            pl.BlockSpec((ep, cap, d), lambda i: (0, i, 0)),
            pl.BlockSpec((1, d, d_ff), lambda i: (i, 0, 0)),
            pl.BlockSpec((1, 1, d_ff), lambda i: (i, 0, 0)),
            pl.BlockSpec((1, d_ff, d), lambda i: (i, 0, 0)),
            pl.BlockSpec((1, 1, d), lambda i: (i, 0, 0)),
        ],
        out_specs=pl.BlockSpec((ep, cap, d), lambda i: (0, i, 0)),
        out_shape=jax.ShapeDtypeStruct((ep, n, d), jnp.float32),
    )(inputs, W1, b1, W2, b2)
